# baseline (speedup 1.0000x reference)
import functools

import jax
import jax.numpy as jnp
from jax import lax
from jax.experimental import pallas as pl
from jax.experimental.pallas import tpu as pltpu
from jax.experimental.pallas import tpu_sc as plsc

M = 2048
NN = 32
LOC = 16
LHID = 128
G = 4
DIN = 64
DOUT = 64
B = 2
E = M * NN
BF = B * DOUT
YW = B * G * DOUT

_EB = 2048


def _mlp_body(maps_ref, w1_ref, b1_ref, w2_ref, b2_ref, c_ref):
    mp = maps_ref[...]
    h = jnp.tanh(
        jnp.dot(mp, w1_ref[...], preferred_element_type=jnp.float32)
        + b1_ref[...])
    cols = []
    for g in range(G):
        hg = h[:, g * LHID:(g + 1) * LHID]
        cg = jnp.sum(hg * w2_ref[g:g + 1, :], axis=1, keepdims=True)
        cols.append(cg + b2_ref[0:1, g:g + 1])
    c_ref[...] = jnp.concatenate(cols, axis=1)


def _mlp(maps_flat, w1cat, b1cat, w2rows, b2row):
    return pl.pallas_call(
        _mlp_body,
        grid=(E // _EB,),
        in_specs=[
            pl.BlockSpec((_EB, LOC), lambda i: (i, 0)),
            pl.BlockSpec((LOC, G * LHID), lambda i: (0, 0)),
            pl.BlockSpec((1, G * LHID), lambda i: (0, 0)),
            pl.BlockSpec((G, LHID), lambda i: (0, 0)),
            pl.BlockSpec((1, G), lambda i: (0, 0)),
        ],
        out_specs=pl.BlockSpec((_EB, G), lambda i: (i, 0)),
        out_shape=jax.ShapeDtypeStruct((E, G), jnp.float32),
    )(maps_flat, w1cat, b1cat, w2rows, b2row)


_MB = 512


def _softmax_body(c_ref, keep_ref, w_ref):
    c = c_ref[...]
    m = jnp.max(c, axis=1, keepdims=True)
    ex = jnp.exp(c - m)
    s = jnp.sum(ex, axis=1, keepdims=True)
    w = (ex / s * keep_ref[...]).reshape(_MB * NN, G)
    row = lax.broadcasted_iota(jnp.int32, (G, G * 16), 0)
    colg = lax.broadcasted_iota(jnp.int32, (G, G * 16), 1) // 16
    k_mat = (row == colg).astype(jnp.float32)
    w_ref[...] = jnp.dot(w, k_mat, preferred_element_type=jnp.float32)


def _softmax_mask(c3, keep3):
    return pl.pallas_call(
        _softmax_body,
        grid=(M // _MB,),
        in_specs=[
            pl.BlockSpec((_MB, NN, G), lambda i: (i, 0, 0)),
            pl.BlockSpec((_MB, NN, 1), lambda i: (i, 0, 0)),
        ],
        out_specs=pl.BlockSpec((_MB * NN, G * 16), lambda i: (i, 0)),
        out_shape=jax.ShapeDtypeStruct((E, G * 16), jnp.float32),
    )(c3, keep3)


_YB = 256


def _y_body(x_ref, w_ref, y_ref):
    pieces = []
    for b in range(B):
        xb = x_ref[b]
        for g in range(G):
            pieces.append(jnp.dot(xb, w_ref[g],
                                  preferred_element_type=jnp.float32))
    y_ref[...] = jnp.concatenate(pieces, axis=1)


def _y_table(x, xe_W):
    return pl.pallas_call(
        _y_body,
        grid=(M // _YB,),
        in_specs=[
            pl.BlockSpec((B, _YB, DIN), lambda i: (0, i, 0)),
            pl.BlockSpec((G, DIN, DOUT), lambda i: (0, 0, 0)),
        ],
        out_specs=pl.BlockSpec((_YB, YW), lambda i: (i, 0)),
        out_shape=jax.ShapeDtypeStruct((M, YW), jnp.float32),
    )(x, xe_W)


_CH = 64
_NTILES = 32
_EPT = E // _NTILES
_NCH = _EPT // _CH
_RPT = M // 16


def _agg_body(lidx_hbm, w_hbm, y_hbm, part_hbm,
              idx_v, col_v, row_v, w_v, rows_v, msg_v, acc_sh,
              gsem0, gsem1, lsem0, lsem1):
    c = lax.axis_index("c")
    s = lax.axis_index("s")
    wid = c * 16 + s
    base = wid * _EPT
    gsems = (gsem0, gsem1)
    lsems = (lsem0, lsem1)

    def zrow(i, carry):
        for j in range(BF // 16):
            msg_v[0, i, pl.ds(j * 16, 16)] = jnp.zeros((16,), jnp.float32)
            msg_v[1, i, pl.ds(j * 16, 16)] = jnp.zeros((16,), jnp.float32)
        return carry
    lax.fori_loop(0, _CH, zrow, 0)
    pltpu.sync_copy(msg_v.at[0], acc_sh.at[pl.ds(s * _RPT, _RPT // 2), :])
    pltpu.sync_copy(msg_v.at[1],
                    acc_sh.at[pl.ds(s * _RPT + _RPT // 2, _RPT // 2), :])

    pltpu.sync_copy(lidx_hbm.at[pl.ds(pl.multiple_of(base, _EPT), _EPT)],
                    idx_v)

    def rcsplit(chi, carry):
        for k in range(_CH // 16):
            iv = idx_v[pl.ds(chi * _CH + k * 16, 16)]
            col_v[pl.ds(chi * _CH + k * 16, 16)] = lax.bitwise_and(iv, M - 1)
            row_v[chi, pl.ds(k * 16, 16)] = lax.shift_right_logical(iv, 11)
        return carry
    lax.fori_loop(0, _NCH, rcsplit, 0)
    plsc.subcore_barrier()

    def _stage(ch, bi):
        off = pl.multiple_of(base + ch * _CH, _CH)
        pltpu.async_copy(w_hbm.at[pl.ds(off, _CH), :], w_v.at[bi], lsems[bi])
        pltpu.async_copy(y_hbm.at[col_v.at[pl.ds(ch * _CH, _CH)]],
                         rows_v.at[bi], gsems[bi])

    def _compute(ch, bi):
        pltpu.make_async_copy(y_hbm.at[col_v.at[pl.ds(0, _CH)]],
                              rows_v.at[bi], gsems[bi]).wait()
        pltpu.make_async_copy(w_hbm.at[pl.ds(0, _CH), :], w_v.at[bi],
                              lsems[bi]).wait()

        def edge(e, carry2):
            ws = [w_v[bi, e, pl.ds(g * 16, 16)] for g in range(G)]
            for b in range(B):
                for k in range(DOUT // 16):
                    acc = ws[0] * rows_v[bi, e, pl.ds(b * G * DOUT + k * 16,
                                                      16)]
                    for g in range(1, G):
                        acc = acc + ws[g] * rows_v[
                            bi, e, pl.ds(b * G * DOUT + g * DOUT + k * 16, 16)]
                    msg_v[bi, e, pl.ds(b * DOUT + k * 16, 16)] = acc
            return carry2
        lax.fori_loop(0, _CH, edge, 0)
        pltpu.sync_copy(msg_v.at[bi], acc_sh.at[row_v.at[ch]], add=True)

    _stage(0, 0)

    def chunk(it, carry):
        ch = it * 2
        _stage(ch + 1, 1)
        _compute(ch, 0)

        @pl.when(ch + 2 < _NCH)
        def _():
            _stage(ch + 2, 0)
        _compute(ch + 1, 1)
        return carry
    lax.fori_loop(0, _NCH // 2, chunk, 0)

    plsc.subcore_barrier()
    pltpu.sync_copy(acc_sh.at[pl.ds(s * _RPT, _RPT), :],
                    part_hbm.at[c, pl.ds(s * _RPT, _RPT), :])


def _aggregate(lidx, w_eg, y):
    mesh = plsc.VectorSubcoreMesh(core_axis_name="c", subcore_axis_name="s")
    agg = functools.partial(
        pl.kernel,
        mesh=mesh,
        out_type=jax.ShapeDtypeStruct((2, M, BF), jnp.float32),
        scratch_types=[
            pltpu.VMEM((_EPT,), jnp.int32),
            pltpu.VMEM((_EPT,), jnp.int32),
            pltpu.VMEM((_NCH, _CH), jnp.int32),
            pltpu.VMEM((2, _CH, G * 16), jnp.float32),
            pltpu.VMEM((2, _CH, YW), jnp.float32),
            pltpu.VMEM((2, _CH, BF), jnp.float32),
            pltpu.VMEM_SHARED((M, BF), jnp.float32),
            pltpu.SemaphoreType.DMA,
            pltpu.SemaphoreType.DMA,
            pltpu.SemaphoreType.DMA,
            pltpu.SemaphoreType.DMA,
        ],
    )(_agg_body)
    return agg(lidx, w_eg, y)


_FB = 256


def _finish_body(p_ref, b_ref, o_ref):
    p = p_ref[...]
    t = p[0] + p[1]
    outs = [t[:, b * DOUT:(b + 1) * DOUT] + b_ref[...] for b in range(B)]
    o_ref[...] = jnp.stack(outs, axis=0)


def _finish(part, bias_row):
    return pl.pallas_call(
        _finish_body,
        grid=(M // _FB,),
        in_specs=[
            pl.BlockSpec((2, _FB, BF), lambda i: (0, i, 0)),
            pl.BlockSpec((1, DOUT), lambda i: (0, 0)),
        ],
        out_specs=pl.BlockSpec((B, _FB, DOUT), lambda i: (0, i, 0)),
        out_shape=jax.ShapeDtypeStruct((B, M, DOUT), jnp.float32),
    )(part, bias_row)


def kernel(x, maps, L_idx, fc1_W, fc1_b, fc2_W, fc2_b, xe_W, xe_b):
    lidx = L_idx.astype(jnp.int32)

    iota = jnp.arange(E, dtype=jnp.int32)
    half = (M * M) // 2
    in_lo = lidx < half
    idx_lo = jnp.where(in_lo, lidx, 0)
    idx_hi = jnp.where(in_lo, 0, lidx - half)
    val_lo = jnp.where(in_lo, iota, -1)
    val_hi = jnp.where(in_lo, -1, iota)
    win_lo = jnp.zeros((half,), jnp.int32).at[idx_lo].max(val_lo)
    win_hi = jnp.zeros((half,), jnp.int32).at[idx_hi].max(val_hi)
    winner = jnp.where(in_lo, win_lo[idx_lo], win_hi[idx_hi])
    keep = (winner == iota).astype(jnp.float32)

    maps_flat = maps.reshape(E, LOC)
    w1cat = jnp.transpose(fc1_W, (1, 0, 2)).reshape(LOC, G * LHID)
    b1cat = fc1_b.reshape(1, G * LHID)
    w2rows = fc2_W.reshape(G, LHID)
    b2row = fc2_b.reshape(1, G)

    c = _mlp(maps_flat, w1cat, b1cat, w2rows, b2row)
    w_exp = _softmax_mask(c.reshape(M, NN, G), keep.reshape(M, NN, 1))

    y = _y_table(x, xe_W)
    part = _aggregate(lidx, w_exp, y)
    return _finish(part, xe_b.reshape(1, DOUT))


# speedup vs baseline: 1.9562x; 1.9562x over previous
import functools

import jax
import jax.numpy as jnp
from jax import lax
from jax.experimental import pallas as pl
from jax.experimental.pallas import tpu as pltpu
from jax.experimental.pallas import tpu_sc as plsc

M = 2048
NN = 32
LOC = 16
LHID = 128
G = 4
DIN = 64
DOUT = 64
B = 2
E = M * NN
BF = B * DOUT
YW = B * G * DOUT

_EB = 2048


def _mlp_body(maps_ref, w1_ref, b1_ref, w2_ref, b2_ref, c_ref):
    mp = maps_ref[...]
    h = jnp.tanh(
        jnp.dot(mp, w1_ref[...], preferred_element_type=jnp.float32)
        + b1_ref[...])
    cols = []
    for g in range(G):
        hg = h[:, g * LHID:(g + 1) * LHID]
        cg = jnp.sum(hg * w2_ref[g:g + 1, :], axis=1, keepdims=True)
        cols.append(cg + b2_ref[0:1, g:g + 1])
    c_ref[...] = jnp.concatenate(cols, axis=1)


def _mlp(maps_flat, w1cat, b1cat, w2rows, b2row):
    return pl.pallas_call(
        _mlp_body,
        grid=(E // _EB,),
        in_specs=[
            pl.BlockSpec((_EB, LOC), lambda i: (i, 0)),
            pl.BlockSpec((LOC, G * LHID), lambda i: (0, 0)),
            pl.BlockSpec((1, G * LHID), lambda i: (0, 0)),
            pl.BlockSpec((G, LHID), lambda i: (0, 0)),
            pl.BlockSpec((1, G), lambda i: (0, 0)),
        ],
        out_specs=pl.BlockSpec((_EB, G), lambda i: (i, 0)),
        out_shape=jax.ShapeDtypeStruct((E, G), jnp.float32),
    )(maps_flat, w1cat, b1cat, w2rows, b2row)


_MB = 512


def _softmax_body(c_ref, keep_ref, w_ref):
    c = c_ref[...]
    m = jnp.max(c, axis=1, keepdims=True)
    ex = jnp.exp(c - m)
    s = jnp.sum(ex, axis=1, keepdims=True)
    w = (ex / s * keep_ref[...]).reshape(_MB * NN, G)
    row = lax.broadcasted_iota(jnp.int32, (G, G * 16), 0)
    colg = lax.broadcasted_iota(jnp.int32, (G, G * 16), 1) // 16
    k_mat = (row == colg).astype(jnp.float32)
    w_ref[...] = jnp.dot(w, k_mat, preferred_element_type=jnp.float32)


def _softmax_mask(c3, keep3):
    return pl.pallas_call(
        _softmax_body,
        grid=(M // _MB,),
        in_specs=[
            pl.BlockSpec((_MB, NN, G), lambda i: (i, 0, 0)),
            pl.BlockSpec((_MB, NN, 1), lambda i: (i, 0, 0)),
        ],
        out_specs=pl.BlockSpec((_MB * NN, G * 16), lambda i: (i, 0)),
        out_shape=jax.ShapeDtypeStruct((E, G * 16), jnp.float32),
    )(c3, keep3)


_YB = 256


def _y_body(x_ref, w_ref, y_ref):
    pieces = []
    for b in range(B):
        xb = x_ref[b]
        for g in range(G):
            pieces.append(jnp.dot(xb, w_ref[g],
                                  preferred_element_type=jnp.float32))
    y_ref[...] = jnp.concatenate(pieces, axis=1)


def _y_table(x, xe_W):
    return pl.pallas_call(
        _y_body,
        grid=(M // _YB,),
        in_specs=[
            pl.BlockSpec((B, _YB, DIN), lambda i: (0, i, 0)),
            pl.BlockSpec((G, DIN, DOUT), lambda i: (0, 0, 0)),
        ],
        out_specs=pl.BlockSpec((_YB, YW), lambda i: (i, 0)),
        out_shape=jax.ShapeDtypeStruct((M, YW), jnp.float32),
    )(x, xe_W)


_CH = 64
_NTILES = 32
_EPT = E // _NTILES
_NCH = _EPT // _CH
_RPT = M // 16


def _agg_body(lidx_hbm, w_hbm, y_hbm, part_hbm,
              idx_v, col_v, row_v, w_v, rows_v, msg_v, acc_sh,
              gsem0, gsem1, lsem0, lsem1):
    c = lax.axis_index("c")
    s = lax.axis_index("s")
    wid = c * 16 + s
    base = wid * _EPT
    gsems = (gsem0, gsem1)
    lsems = (lsem0, lsem1)

    def zrow(i, carry):
        for j in range(BF // 16):
            msg_v[0, i, pl.ds(j * 16, 16)] = jnp.zeros((16,), jnp.float32)
            msg_v[1, i, pl.ds(j * 16, 16)] = jnp.zeros((16,), jnp.float32)
        return carry
    lax.fori_loop(0, _CH, zrow, 0)
    pltpu.sync_copy(msg_v.at[0], acc_sh.at[pl.ds(s * _RPT, _RPT // 2), :])
    pltpu.sync_copy(msg_v.at[1],
                    acc_sh.at[pl.ds(s * _RPT + _RPT // 2, _RPT // 2), :])

    pltpu.sync_copy(lidx_hbm.at[pl.ds(pl.multiple_of(base, _EPT), _EPT)],
                    idx_v)

    def rcsplit(chi, carry):
        for k in range(_CH // 16):
            iv = idx_v[pl.ds(chi * _CH + k * 16, 16)]
            col_v[pl.ds(chi * _CH + k * 16, 16)] = lax.bitwise_and(iv, M - 1)
            row_v[chi, pl.ds(k * 16, 16)] = lax.shift_right_logical(iv, 11)
        return carry
    lax.fori_loop(0, _NCH, rcsplit, 0)
    plsc.subcore_barrier()

    def _stage(ch, bi):
        off = pl.multiple_of(base + ch * _CH, _CH)
        pltpu.async_copy(w_hbm.at[pl.ds(off, _CH), :], w_v.at[bi], lsems[bi])
        pltpu.async_copy(y_hbm.at[col_v.at[pl.ds(ch * _CH, _CH)]],
                         rows_v.at[bi], gsems[bi])

    def _compute(ch, bi):
        pltpu.make_async_copy(y_hbm.at[col_v.at[pl.ds(0, _CH)]],
                              rows_v.at[bi], gsems[bi]).wait()
        pltpu.make_async_copy(w_hbm.at[pl.ds(0, _CH), :], w_v.at[bi],
                              lsems[bi]).wait()

        @plsc.parallel_loop(0, _CH, 1, unroll=4)
        def edge(e):
            ws = [w_v[bi, e, pl.ds(g * 16, 16)] for g in range(G)]
            for b in range(B):
                for k in range(DOUT // 16):
                    acc = ws[0] * rows_v[bi, e, pl.ds(b * G * DOUT + k * 16,
                                                      16)]
                    for g in range(1, G):
                        acc = acc + ws[g] * rows_v[
                            bi, e, pl.ds(b * G * DOUT + g * DOUT + k * 16, 16)]
                    msg_v[bi, e, pl.ds(b * DOUT + k * 16, 16)] = acc
        pltpu.sync_copy(msg_v.at[bi], acc_sh.at[row_v.at[ch]], add=True)

    _stage(0, 0)

    def chunk(it, carry):
        ch = it * 2
        _stage(ch + 1, 1)
        _compute(ch, 0)

        @pl.when(ch + 2 < _NCH)
        def _():
            _stage(ch + 2, 0)
        _compute(ch + 1, 1)
        return carry
    lax.fori_loop(0, _NCH // 2, chunk, 0)

    plsc.subcore_barrier()
    pltpu.sync_copy(acc_sh.at[pl.ds(s * _RPT, _RPT), :],
                    part_hbm.at[c, pl.ds(s * _RPT, _RPT), :])


def _aggregate(lidx, w_eg, y):
    mesh = plsc.VectorSubcoreMesh(core_axis_name="c", subcore_axis_name="s")
    agg = functools.partial(
        pl.kernel,
        mesh=mesh,
        out_type=jax.ShapeDtypeStruct((2, M, BF), jnp.float32),
        scratch_types=[
            pltpu.VMEM((_EPT,), jnp.int32),
            pltpu.VMEM((_EPT,), jnp.int32),
            pltpu.VMEM((_NCH, _CH), jnp.int32),
            pltpu.VMEM((2, _CH, G * 16), jnp.float32),
            pltpu.VMEM((2, _CH, YW), jnp.float32),
            pltpu.VMEM((2, _CH, BF), jnp.float32),
            pltpu.VMEM_SHARED((M, BF), jnp.float32),
            pltpu.SemaphoreType.DMA,
            pltpu.SemaphoreType.DMA,
            pltpu.SemaphoreType.DMA,
            pltpu.SemaphoreType.DMA,
        ],
    )(_agg_body)
    return agg(lidx, w_eg, y)


_FB = 256


def _finish_body(p_ref, b_ref, o_ref):
    p = p_ref[...]
    t = p[0] + p[1]
    outs = [t[:, b * DOUT:(b + 1) * DOUT] + b_ref[...] for b in range(B)]
    o_ref[...] = jnp.stack(outs, axis=0)


def _finish(part, bias_row):
    return pl.pallas_call(
        _finish_body,
        grid=(M // _FB,),
        in_specs=[
            pl.BlockSpec((2, _FB, BF), lambda i: (0, i, 0)),
            pl.BlockSpec((1, DOUT), lambda i: (0, 0)),
        ],
        out_specs=pl.BlockSpec((B, _FB, DOUT), lambda i: (0, i, 0)),
        out_shape=jax.ShapeDtypeStruct((B, M, DOUT), jnp.float32),
    )(part, bias_row)


def kernel(x, maps, L_idx, fc1_W, fc1_b, fc2_W, fc2_b, xe_W, xe_b):
    lidx = L_idx.astype(jnp.int32)

    iota = jnp.arange(E, dtype=jnp.int32)
    winner = jnp.zeros((M * M,), jnp.int32).at[lidx].max(iota)
    keep = (winner[lidx] == iota).astype(jnp.float32)

    maps_flat = maps.reshape(E, LOC)
    w1cat = jnp.transpose(fc1_W, (1, 0, 2)).reshape(LOC, G * LHID)
    b1cat = fc1_b.reshape(1, G * LHID)
    w2rows = fc2_W.reshape(G, LHID)
    b2row = fc2_b.reshape(1, G)

    c = _mlp(maps_flat, w1cat, b1cat, w2rows, b2row)
    w_exp = _softmax_mask(c.reshape(M, NN, G), keep.reshape(M, NN, 1))

    y = _y_table(x, xe_W)
    part = _aggregate(lidx, w_exp, y)
    return _finish(part, xe_b.reshape(1, DOUT))


# speedup vs baseline: 2.1623x; 1.1054x over previous
import functools

import jax
import jax.numpy as jnp
from jax import lax
from jax.experimental import pallas as pl
from jax.experimental.pallas import tpu as pltpu
from jax.experimental.pallas import tpu_sc as plsc

M = 2048
NN = 32
LOC = 16
LHID = 128
G = 4
DIN = 64
DOUT = 64
B = 2
E = M * NN
BF = B * DOUT
YW = B * G * DOUT

_EB = 2048


def _mlp_body(maps_ref, w1_ref, b1_ref, w2_ref, b2_ref, c_ref):
    mp = maps_ref[...]
    h = jnp.tanh(
        jnp.dot(mp, w1_ref[...], preferred_element_type=jnp.float32)
        + b1_ref[...])
    cols = []
    for g in range(G):
        hg = h[:, g * LHID:(g + 1) * LHID]
        cg = jnp.sum(hg * w2_ref[g:g + 1, :], axis=1, keepdims=True)
        cols.append(cg + b2_ref[0:1, g:g + 1])
    c_ref[...] = jnp.concatenate(cols, axis=1)


def _mlp(maps_flat, w1cat, b1cat, w2rows, b2row):
    return pl.pallas_call(
        _mlp_body,
        grid=(E // _EB,),
        in_specs=[
            pl.BlockSpec((_EB, LOC), lambda i: (i, 0)),
            pl.BlockSpec((LOC, G * LHID), lambda i: (0, 0)),
            pl.BlockSpec((1, G * LHID), lambda i: (0, 0)),
            pl.BlockSpec((G, LHID), lambda i: (0, 0)),
            pl.BlockSpec((1, G), lambda i: (0, 0)),
        ],
        out_specs=pl.BlockSpec((_EB, G), lambda i: (i, 0)),
        out_shape=jax.ShapeDtypeStruct((E, G), jnp.float32),
    )(maps_flat, w1cat, b1cat, w2rows, b2row)


_MB = 512


def _softmax_body(c_ref, w_ref):
    c = c_ref[...]
    m = jnp.max(c, axis=1, keepdims=True)
    ex = jnp.exp(c - m)
    s = jnp.sum(ex, axis=1, keepdims=True)
    w = (ex / s).reshape(_MB * NN, G)
    row = lax.broadcasted_iota(jnp.int32, (G, G * 16), 0)
    colg = lax.broadcasted_iota(jnp.int32, (G, G * 16), 1) // 16
    k_mat = (row == colg).astype(jnp.float32)
    w_ref[...] = jnp.dot(w, k_mat, preferred_element_type=jnp.float32)


def _softmax_mask(c3):
    return pl.pallas_call(
        _softmax_body,
        grid=(M // _MB,),
        in_specs=[
            pl.BlockSpec((_MB, NN, G), lambda i: (i, 0, 0)),
        ],
        out_specs=pl.BlockSpec((_MB * NN, G * 16), lambda i: (i, 0)),
        out_shape=jax.ShapeDtypeStruct((E, G * 16), jnp.float32),
    )(c3)


_YB = 256


def _y_body(x_ref, w_ref, y_ref):
    pieces = []
    for b in range(B):
        xb = x_ref[b]
        for g in range(G):
            pieces.append(jnp.dot(xb, w_ref[g],
                                  preferred_element_type=jnp.float32))
    y_ref[...] = jnp.concatenate(pieces, axis=1)


def _y_table(x, xe_W):
    return pl.pallas_call(
        _y_body,
        grid=(M // _YB,),
        in_specs=[
            pl.BlockSpec((B, _YB, DIN), lambda i: (0, i, 0)),
            pl.BlockSpec((G, DIN, DOUT), lambda i: (0, 0, 0)),
        ],
        out_specs=pl.BlockSpec((_YB, YW), lambda i: (i, 0)),
        out_shape=jax.ShapeDtypeStruct((M, YW), jnp.float32),
    )(x, xe_W)


_CH = 64
_NTILES = 32
_EPT = E // _NTILES
_NCH = _EPT // _CH
_RPT = M // 16


def _agg_body(lidx_hbm, w_hbm, y_hbm, win_hbm, part_hbm,
              idx_v, col_v, row_v, w_v, rows_v, msg_v, win_v, acc_sh,
              gsem0, gsem1, lsem0, lsem1, wsem0, wsem1):
    c = lax.axis_index("c")
    s = lax.axis_index("s")
    wid = c * 16 + s
    base = wid * _EPT
    gsems = (gsem0, gsem1)
    lsems = (lsem0, lsem1)
    wsems = (wsem0, wsem1)

    def zrow(i, carry):
        for j in range(BF // 16):
            msg_v[0, i, pl.ds(j * 16, 16)] = jnp.zeros((16,), jnp.float32)
            msg_v[1, i, pl.ds(j * 16, 16)] = jnp.zeros((16,), jnp.float32)
        return carry
    lax.fori_loop(0, _CH, zrow, 0)
    pltpu.sync_copy(msg_v.at[0], acc_sh.at[pl.ds(s * _RPT, _RPT // 2), :])
    pltpu.sync_copy(msg_v.at[1],
                    acc_sh.at[pl.ds(s * _RPT + _RPT // 2, _RPT // 2), :])

    pltpu.sync_copy(lidx_hbm.at[pl.ds(pl.multiple_of(base, _EPT), _EPT)],
                    idx_v)

    def rcsplit(chi, carry):
        for k in range(_CH // 16):
            iv = idx_v[pl.ds(chi * _CH + k * 16, 16)]
            col_v[pl.ds(chi * _CH + k * 16, 16)] = lax.bitwise_and(iv, M - 1)
        return carry
    lax.fori_loop(0, _NCH, rcsplit, 0)
    plsc.subcore_barrier()

    def _stage(ch, bi):
        off = pl.multiple_of(base + ch * _CH, _CH)
        pltpu.async_copy(w_hbm.at[pl.ds(off, _CH), :], w_v.at[bi], lsems[bi])
        pltpu.async_copy(win_hbm.at[idx_v.at[pl.ds(ch * _CH, _CH)]],
                         win_v.at[bi], wsems[bi])
        pltpu.async_copy(y_hbm.at[col_v.at[pl.ds(ch * _CH, _CH)]],
                         rows_v.at[bi], gsems[bi])

    def _compute(ch, bi):
        pltpu.make_async_copy(y_hbm.at[col_v.at[pl.ds(0, _CH)]],
                              rows_v.at[bi], gsems[bi]).wait()
        pltpu.make_async_copy(w_hbm.at[pl.ds(0, _CH), :], w_v.at[bi],
                              lsems[bi]).wait()
        pltpu.make_async_copy(win_hbm.at[pl.ds(0, _CH)], win_v.at[bi],
                              wsems[bi]).wait()

        for k in range(_CH // 16):
            iv = idx_v[pl.ds(ch * _CH + k * 16, 16)]
            rw = lax.shift_right_logical(iv, 11)
            eid = base + ch * _CH + k * 16 + jnp.arange(16, dtype=jnp.int32)
            wn = win_v[bi, pl.ds(k * 16, 16)]
            row_v[ch, pl.ds(k * 16, 16)] = jnp.where(wn == eid, rw, M)

        @plsc.parallel_loop(0, _CH, 1, unroll=4)
        def edge(e):
            ws = [w_v[bi, e, pl.ds(g * 16, 16)] for g in range(G)]
            for b in range(B):
                for k in range(DOUT // 16):
                    acc = ws[0] * rows_v[bi, e, pl.ds(b * G * DOUT + k * 16,
                                                      16)]
                    for g in range(1, G):
                        acc = acc + ws[g] * rows_v[
                            bi, e, pl.ds(b * G * DOUT + g * DOUT + k * 16, 16)]
                    msg_v[bi, e, pl.ds(b * DOUT + k * 16, 16)] = acc
        pltpu.sync_copy(msg_v.at[bi], acc_sh.at[row_v.at[ch]], add=True)

    _stage(0, 0)

    def chunk(it, carry):
        ch = it * 2
        _stage(ch + 1, 1)
        _compute(ch, 0)

        @pl.when(ch + 2 < _NCH)
        def _():
            _stage(ch + 2, 0)
        _compute(ch + 1, 1)
        return carry
    lax.fori_loop(0, _NCH // 2, chunk, 0)

    plsc.subcore_barrier()
    pltpu.sync_copy(acc_sh.at[pl.ds(s * _RPT, _RPT), :],
                    part_hbm.at[c, pl.ds(s * _RPT, _RPT), :])


def _aggregate(lidx, w_eg, y, winner):
    mesh = plsc.VectorSubcoreMesh(core_axis_name="c", subcore_axis_name="s")
    agg = functools.partial(
        pl.kernel,
        mesh=mesh,
        out_type=jax.ShapeDtypeStruct((2, M, BF), jnp.float32),
        scratch_types=[
            pltpu.VMEM((_EPT,), jnp.int32),
            pltpu.VMEM((_EPT,), jnp.int32),
            pltpu.VMEM((_NCH, _CH), jnp.int32),
            pltpu.VMEM((2, _CH, G * 16), jnp.float32),
            pltpu.VMEM((2, _CH, YW), jnp.float32),
            pltpu.VMEM((2, _CH, BF), jnp.float32),
            pltpu.VMEM((2, _CH), jnp.int32),
            pltpu.VMEM_SHARED((M + 16, BF), jnp.float32),
            pltpu.SemaphoreType.DMA,
            pltpu.SemaphoreType.DMA,
            pltpu.SemaphoreType.DMA,
            pltpu.SemaphoreType.DMA,
            pltpu.SemaphoreType.DMA,
            pltpu.SemaphoreType.DMA,
        ],
    )(_agg_body)
    return agg(lidx, w_eg, y, winner)


_FB = 256


def _finish_body(p_ref, b_ref, o_ref):
    p = p_ref[...]
    t = p[0] + p[1]
    outs = [t[:, b * DOUT:(b + 1) * DOUT] + b_ref[...] for b in range(B)]
    o_ref[...] = jnp.stack(outs, axis=0)


def _finish(part, bias_row):
    return pl.pallas_call(
        _finish_body,
        grid=(M // _FB,),
        in_specs=[
            pl.BlockSpec((2, _FB, BF), lambda i: (0, i, 0)),
            pl.BlockSpec((1, DOUT), lambda i: (0, 0)),
        ],
        out_specs=pl.BlockSpec((B, _FB, DOUT), lambda i: (0, i, 0)),
        out_shape=jax.ShapeDtypeStruct((B, M, DOUT), jnp.float32),
    )(part, bias_row)


def kernel(x, maps, L_idx, fc1_W, fc1_b, fc2_W, fc2_b, xe_W, xe_b):
    lidx = L_idx.astype(jnp.int32)

    iota = jnp.arange(E, dtype=jnp.int32)
    winner = jnp.zeros((M * M,), jnp.int32).at[lidx].max(iota)

    maps_flat = maps.reshape(E, LOC)
    w1cat = jnp.transpose(fc1_W, (1, 0, 2)).reshape(LOC, G * LHID)
    b1cat = fc1_b.reshape(1, G * LHID)
    w2rows = fc2_W.reshape(G, LHID)
    b2row = fc2_b.reshape(1, G)

    c = _mlp(maps_flat, w1cat, b1cat, w2rows, b2row)
    w_exp = _softmax_mask(c.reshape(M, NN, G))

    y = _y_table(x, xe_W)
    part = _aggregate(lidx, w_exp, y, winner)
    return _finish(part, xe_b.reshape(1, DOUT))


# speedup vs baseline: 2.5669x; 1.1871x over previous
import functools

import jax
import jax.numpy as jnp
from jax import lax
from jax.experimental import pallas as pl
from jax.experimental.pallas import tpu as pltpu
from jax.experimental.pallas import tpu_sc as plsc

M = 2048
NN = 32
LOC = 16
LHID = 128
G = 4
DIN = 64
DOUT = 64
B = 2
E = M * NN
BF = B * DOUT
YW = B * G * DOUT

_EB = 2048


def _attn_body(maps_ref, w1_ref, b1_ref, w2_ref, b2_ref, w_ref):
    mp = maps_ref[...]
    h = jnp.tanh(
        jnp.dot(mp, w1_ref[...], preferred_element_type=jnp.float32)
        + b1_ref[...])
    cols = []
    for g in range(G):
        hg = h[:, g * LHID:(g + 1) * LHID]
        cg = jnp.sum(hg * w2_ref[g:g + 1, :], axis=1, keepdims=True)
        cols.append(cg + b2_ref[0:1, g:g + 1])
    c3 = jnp.concatenate(cols, axis=1).reshape(_EB // NN, NN, G)
    m = jnp.max(c3, axis=1, keepdims=True)
    ex = jnp.exp(c3 - m)
    s = jnp.sum(ex, axis=1, keepdims=True)
    w = (ex / s).reshape(_EB, G)
    row = lax.broadcasted_iota(jnp.int32, (G, G * 16), 0)
    colg = lax.broadcasted_iota(jnp.int32, (G, G * 16), 1) // 16
    k_mat = (row == colg).astype(jnp.float32)
    w_ref[...] = jnp.dot(w, k_mat, preferred_element_type=jnp.float32)


def _attn(maps_flat, w1cat, b1cat, w2rows, b2row):
    return pl.pallas_call(
        _attn_body,
        grid=(E // _EB,),
        in_specs=[
            pl.BlockSpec((_EB, LOC), lambda i: (i, 0)),
            pl.BlockSpec((LOC, G * LHID), lambda i: (0, 0)),
            pl.BlockSpec((1, G * LHID), lambda i: (0, 0)),
            pl.BlockSpec((G, LHID), lambda i: (0, 0)),
            pl.BlockSpec((1, G), lambda i: (0, 0)),
        ],
        out_specs=pl.BlockSpec((_EB, G * 16), lambda i: (i, 0)),
        out_shape=jax.ShapeDtypeStruct((E, G * 16), jnp.float32),
    )(maps_flat, w1cat, b1cat, w2rows, b2row)


_YB = 256


def _y_body(x_ref, w_ref, y_ref):
    pieces = []
    for b in range(B):
        xb = x_ref[b]
        for g in range(G):
            pieces.append(jnp.dot(xb, w_ref[g],
                                  preferred_element_type=jnp.float32))
    y_ref[...] = jnp.concatenate(pieces, axis=1)


def _y_table(x, xe_W):
    return pl.pallas_call(
        _y_body,
        grid=(M // _YB,),
        in_specs=[
            pl.BlockSpec((B, _YB, DIN), lambda i: (0, i, 0)),
            pl.BlockSpec((G, DIN, DOUT), lambda i: (0, 0, 0)),
        ],
        out_specs=pl.BlockSpec((_YB, YW), lambda i: (i, 0)),
        out_shape=jax.ShapeDtypeStruct((M, YW), jnp.float32),
    )(x, xe_W)


_CH = 64
_NTILES = 32
_EPT = E // _NTILES
_NCH = _EPT // _CH
_RPT = M // 16


def _agg_body(lidx_hbm, w_hbm, y_hbm, win_hbm, part_hbm,
              idx_v, col_v, row_v, w_v, rows_v, msg_v, win_v, acc_sh,
              gsem0, gsem1, lsem0, lsem1, wsem0, wsem1, ssem0, ssem1):
    c = lax.axis_index("c")
    s = lax.axis_index("s")
    wid = c * 16 + s
    base = wid * _EPT
    gsems = (gsem0, gsem1)
    lsems = (lsem0, lsem1)
    wsems = (wsem0, wsem1)
    ssems = (ssem0, ssem1)

    def zrow(i, carry):
        for j in range(BF // 16):
            msg_v[0, i, pl.ds(j * 16, 16)] = jnp.zeros((16,), jnp.float32)
            msg_v[1, i, pl.ds(j * 16, 16)] = jnp.zeros((16,), jnp.float32)
        return carry
    lax.fori_loop(0, _CH, zrow, 0)
    pltpu.sync_copy(msg_v.at[0], acc_sh.at[pl.ds(s * _RPT, _RPT // 2), :])
    pltpu.sync_copy(msg_v.at[1],
                    acc_sh.at[pl.ds(s * _RPT + _RPT // 2, _RPT // 2), :])

    pltpu.sync_copy(lidx_hbm.at[pl.ds(pl.multiple_of(base, _EPT), _EPT)],
                    idx_v)

    def rcsplit(chi, carry):
        for k in range(_CH // 16):
            iv = idx_v[pl.ds(chi * _CH + k * 16, 16)]
            col_v[pl.ds(chi * _CH + k * 16, 16)] = lax.bitwise_and(iv, M - 1)
        return carry
    lax.fori_loop(0, _NCH, rcsplit, 0)
    plsc.subcore_barrier()

    def _stage(ch, bi):
        off = pl.multiple_of(base + ch * _CH, _CH)
        pltpu.async_copy(w_hbm.at[pl.ds(off, _CH), :], w_v.at[bi], lsems[bi])
        pltpu.async_copy(win_hbm.at[idx_v.at[pl.ds(ch * _CH, _CH)]],
                         win_v.at[bi], wsems[bi])
        pltpu.async_copy(y_hbm.at[col_v.at[pl.ds(ch * _CH, _CH)]],
                         rows_v.at[bi], gsems[bi])

    def _compute(ch, bi):
        pltpu.make_async_copy(y_hbm.at[col_v.at[pl.ds(0, _CH)]],
                              rows_v.at[bi], gsems[bi]).wait()
        pltpu.make_async_copy(w_hbm.at[pl.ds(0, _CH), :], w_v.at[bi],
                              lsems[bi]).wait()
        pltpu.make_async_copy(win_hbm.at[pl.ds(0, _CH)], win_v.at[bi],
                              wsems[bi]).wait()

        for k in range(_CH // 16):
            iv = idx_v[pl.ds(ch * _CH + k * 16, 16)]
            rw = lax.shift_right_logical(iv, 11)
            eid = base + ch * _CH + k * 16 + jnp.arange(16, dtype=jnp.int32)
            wn = win_v[bi, pl.ds(k * 16, 16)]
            row_v[ch, pl.ds(k * 16, 16)] = jnp.where(wn == eid, rw, M)

        @plsc.parallel_loop(0, _CH, 1, unroll=8)
        def edge(e):
            ws = [w_v[bi, e, pl.ds(g * 16, 16)] for g in range(G)]
            for b in range(B):
                for k in range(DOUT // 16):
                    acc = ws[0] * rows_v[bi, e, pl.ds(b * G * DOUT + k * 16,
                                                      16)]
                    for g in range(1, G):
                        acc = acc + ws[g] * rows_v[
                            bi, e, pl.ds(b * G * DOUT + g * DOUT + k * 16, 16)]
                    msg_v[bi, e, pl.ds(b * DOUT + k * 16, 16)] = acc
        pltpu.async_copy(msg_v.at[bi], acc_sh.at[row_v.at[ch]], ssems[bi],
                         add=True)

    def _scatter_wait(bi):
        pltpu.make_async_copy(msg_v.at[bi], acc_sh.at[row_v.at[0]],
                              ssems[bi]).wait()

    _stage(0, 0)

    def chunk(it, carry):
        ch = it * 2
        _stage(ch + 1, 1)

        @pl.when(ch >= 2)
        def _():
            _scatter_wait(0)
        _compute(ch, 0)

        @pl.when(ch + 2 < _NCH)
        def _():
            _stage(ch + 2, 0)

        @pl.when(ch >= 2)
        def _():
            _scatter_wait(1)
        _compute(ch + 1, 1)
        return carry
    lax.fori_loop(0, _NCH // 2, chunk, 0)

    _scatter_wait(0)
    _scatter_wait(1)
    plsc.subcore_barrier()
    pltpu.sync_copy(acc_sh.at[pl.ds(s * _RPT, _RPT), :],
                    part_hbm.at[c, pl.ds(s * _RPT, _RPT), :])


def _aggregate(lidx, w_eg, y, winner):
    mesh = plsc.VectorSubcoreMesh(core_axis_name="c", subcore_axis_name="s")
    agg = functools.partial(
        pl.kernel,
        mesh=mesh,
        out_type=jax.ShapeDtypeStruct((2, M, BF), jnp.float32),
        scratch_types=[
            pltpu.VMEM((_EPT,), jnp.int32),
            pltpu.VMEM((_EPT,), jnp.int32),
            pltpu.VMEM((_NCH, _CH), jnp.int32),
            pltpu.VMEM((2, _CH, G * 16), jnp.float32),
            pltpu.VMEM((2, _CH, YW), jnp.float32),
            pltpu.VMEM((2, _CH, BF), jnp.float32),
            pltpu.VMEM((2, _CH), jnp.int32),
            pltpu.VMEM_SHARED((M + 16, BF), jnp.float32),
            pltpu.SemaphoreType.DMA,
            pltpu.SemaphoreType.DMA,
            pltpu.SemaphoreType.DMA,
            pltpu.SemaphoreType.DMA,
            pltpu.SemaphoreType.DMA,
            pltpu.SemaphoreType.DMA,
            pltpu.SemaphoreType.DMA,
            pltpu.SemaphoreType.DMA,
        ],
    )(_agg_body)
    return agg(lidx, w_eg, y, winner)


_FB = 256


def _finish_body(p_ref, b_ref, o_ref):
    p = p_ref[...]
    t = p[0] + p[1]
    outs = [t[:, b * DOUT:(b + 1) * DOUT] + b_ref[...] for b in range(B)]
    o_ref[...] = jnp.stack(outs, axis=0)


def _finish(part, bias_row):
    return pl.pallas_call(
        _finish_body,
        grid=(M // _FB,),
        in_specs=[
            pl.BlockSpec((2, _FB, BF), lambda i: (0, i, 0)),
            pl.BlockSpec((1, DOUT), lambda i: (0, 0)),
        ],
        out_specs=pl.BlockSpec((B, _FB, DOUT), lambda i: (0, i, 0)),
        out_shape=jax.ShapeDtypeStruct((B, M, DOUT), jnp.float32),
    )(part, bias_row)


def kernel(x, maps, L_idx, fc1_W, fc1_b, fc2_W, fc2_b, xe_W, xe_b):
    lidx = L_idx.astype(jnp.int32)

    iota = jnp.arange(E, dtype=jnp.int32)
    winner = jnp.zeros((M * M,), jnp.int32).at[lidx].max(iota)

    maps_flat = maps.reshape(E, LOC)
    w1cat = jnp.transpose(fc1_W, (1, 0, 2)).reshape(LOC, G * LHID)
    b1cat = fc1_b.reshape(1, G * LHID)
    w2rows = fc2_W.reshape(G, LHID)
    b2row = fc2_b.reshape(1, G)

    w_exp = _attn(maps_flat, w1cat, b1cat, w2rows, b2row)
    y = _y_table(x, xe_W)
    part = _aggregate(lidx, w_exp, y, winner)
    return _finish(part, xe_b.reshape(1, DOUT))
